# baseline (speedup 1.0000x reference)
import functools

import jax
import jax.numpy as jnp
from jax import lax
from jax.experimental import pallas as pl
from jax.experimental.pallas import tpu as pltpu
from jax.experimental.pallas import tpu_sc as plsc

N = 50000
E = 800000
H = 64
B = 512

NC = 2
NS = 16
NW = NC * NS
QN = N // 4
QP = QN // 2
DUMQ = QP
RPTQ = 392
NPADQ = NS * RPTQ
EPW = E // NW
EPWP = EPW + 8
PB = 4096
CAP = 32768
CHK = 128
BLKE = 1024

ROWBLK = 2000
NSTEPS = N // ROWBLK


_SC_MESH = plsc.VectorSubcoreMesh(core_axis_name="c", subcore_axis_name="s",
                                  num_cores=NC, num_subcores=NS)


def _part_body(src_hbm, dst_hbm, gi_hbm, pr_hbm, nb_hbm,
               se_v, de_v, sg0, sg1, sg2, sg3, sp0, sp1, sp2, sp3,
               dg_v, dp_v, nb_v):
  c = lax.axis_index("c")
  s = lax.axis_index("s")
  wid = c * NS + s
  ebase = wid * EPW
  elim = ebase + EPW
  stg = ((sg0, sp0), (sg1, sp1), (sg2, sp2), (sg3, sp3))

  zi = jnp.zeros((16,), jnp.int32)
  dq = zi + DUMQ
  def _dfill(k, _):
    dg_v[pl.ds(k * 16, 16)] = zi
    dp_v[pl.ds(k * 16, 16)] = dq
    return _
  lax.fori_loop(0, BLKE // 16, _dfill, None)

  iota = jnp.arange(16, dtype=jnp.int32)
  curs = (jnp.int32(0),) * 4

  def _block(boff, nedges, curs):
    eofs = pl.multiple_of(ebase + boff, 8)
    pltpu.sync_copy(src_hbm.at[pl.ds(eofs, nedges)],
                    se_v.at[pl.ds(0, nedges)])
    pltpu.sync_copy(dst_hbm.at[pl.ds(eofs, nedges)],
                    de_v.at[pl.ds(0, nedges)])

    def _vreg(r, lcurs):
      sl = pl.ds(r * 16, 16)
      srcv = se_v[sl]
      dstv = de_v[sl]
      valid = (iota - (elim - (ebase + boff + r * 16))) < 0
      giv = srcv + (dstv & 1) * N
      out = []
      for q in range(4):
        dlq = dstv - q * QN
        m = (dlq >= 0) & (dlq < QN) & valid
        lq = lcurs[q]
        cum = plsc.cumsum(m.astype(jnp.int32))
        pos = cum + (lq - 1)
        plsc.store_scatter(stg[q][0], [pos], giv, mask=m)
        plsc.store_scatter(stg[q][1], [pos], dlq >> 1, mask=m)
        out.append(lq + jnp.max(cum))
      return tuple(out)

    lcurs = lax.fori_loop(0, nedges // 16, _vreg, (jnp.int32(0),) * 4)
    new_curs = []
    for q in range(4):
      lq = lcurs[q]
      plsc.store_scatter(stg[q][0], [iota + lq], zi, mask=iota < 16)
      plsc.store_scatter(stg[q][1], [iota + lq], dq, mask=iota < 16)
      pad8 = (lq + 7) & -8
      cur = curs[q]
      lbase = (q * NW + wid) * CAP
      ofs = pl.multiple_of(lbase + cur, 8)
      pltpu.sync_copy(stg[q][0].at[pl.ds(0, PB)], gi_hbm.at[pl.ds(ofs, PB)])
      pltpu.sync_copy(stg[q][1].at[pl.ds(0, PB)], pr_hbm.at[pl.ds(ofs, PB)])
      new_curs.append(pl.multiple_of(cur + pad8, 8))
    return tuple(new_curs)

  for bi in range(6):
    curs = _block(bi * PB, PB, curs)
  curs = _block(6 * PB, EPWP - 6 * PB, curs)

  for q in range(4):
    cur = curs[q]
    lbase = (q * NW + wid) * CAP
    ofs = pl.multiple_of(lbase + cur, 8)
    pltpu.sync_copy(dg_v, gi_hbm.at[pl.ds(ofs, BLKE)])
    pltpu.sync_copy(dp_v, pr_hbm.at[pl.ds(ofs, BLKE)])
    nb_v[...] = zi + ((cur + BLKE - 1) >> 10)
    nofs = pl.multiple_of((q * NW + wid) * 16, 8)
    pltpu.sync_copy(nb_v, nb_hbm.at[pl.ds(nofs, 16)])


_sc_partition = functools.partial(
    pl.kernel,
    out_type=[
        jax.ShapeDtypeStruct((4 * NW * CAP,), jnp.int32),
        jax.ShapeDtypeStruct((4 * NW * CAP,), jnp.int32),
        jax.ShapeDtypeStruct((4 * NW * 16,), jnp.int32),
    ],
    mesh=_SC_MESH,
    compiler_params=pltpu.CompilerParams(needs_layout_passes=False),
    scratch_types=(
        [pltpu.VMEM((PB,), jnp.int32)] * 2
        + [pltpu.VMEM((PB + 16,), jnp.int32)] * 8
        + [pltpu.VMEM((BLKE,), jnp.int32)] * 2
        + [pltpu.VMEM((16,), jnp.int32)]
    ),
)(_part_body)


def _scat_body(h_hbm, gi_hbm, pr_hbm, nb_hbm, agg_hbm,
               acc, gi_f, pr_f, gic0, gic1, gic2, prc0, prc1, prc2,
               r0, r1, r2, nb_v,
               sg0, sg1, sg2, ss0, ss1, ss2):
  c = lax.axis_index("c")
  s = lax.axis_index("s")
  gic = (gic0, gic1, gic2)
  prc = (prc0, prc1, prc2)
  rows = (r0, r1, r2)
  sg = (sg0, sg1, sg2)
  ss = (ss0, ss1, ss2)

  for p in range(2):
    q = 2 * c + p
    zf = jnp.zeros((16,), jnp.float32)
    def _zfill(k, _):
      r0[k // 8, pl.ds((k % 8) * 16, 16)] = zf
      return _
    lax.fori_loop(0, CHK * 8, _zfill, None)
    zbase = s * RPTQ
    for piece in range(3):
      pltpu.sync_copy(r0, acc.at[pl.ds(zbase + piece * CHK, CHK)])
    pltpu.sync_copy(r0.at[pl.ds(0, 8)], acc.at[pl.ds(zbase + 3 * CHK, 8)])
    plsc.subcore_barrier()

    for li_off in (0, NS):
      li = s + li_off
      nofs = pl.multiple_of((q * NW + li) * 16, 8)
      pltpu.sync_copy(nb_hbm.at[pl.ds(nofs, 16)], nb_v)
      nblk = jnp.max(nb_v[...])
      lbase = (q * NW + li) * CAP

      def _blk(b, _):
        bofs = pl.multiple_of(lbase + b * BLKE, 8)
        pltpu.sync_copy(gi_hbm.at[pl.ds(bofs, BLKE)], gi_f)
        pltpu.sync_copy(pr_hbm.at[pl.ds(bofs, BLKE)], pr_f)
        for j in range(8):
          k = j % 3
          if j >= 3:
            pltpu.make_async_copy(rows[k], acc.at[prc[k]], ss[k]).wait()
          for t in range(8):
            tsl = pl.ds(t * 16, 16)
            gic[k][tsl] = gi_f[pl.ds(j * CHK + t * 16, 16)]
            prc[k][tsl] = pr_f[pl.ds(j * CHK + t * 16, 16)]
          pltpu.async_copy(h_hbm.at[gic[k]], rows[k], sg[k])
          if j >= 1:
            km1 = (j - 1) % 3
            pltpu.make_async_copy(h_hbm.at[gic[km1]], rows[km1],
                                  sg[km1]).wait()
            pltpu.async_copy(rows[km1], acc.at[prc[km1]], ss[km1], add=True)
        pltpu.make_async_copy(h_hbm.at[gic[1]], rows[1], sg[1]).wait()
        pltpu.async_copy(rows[1], acc.at[prc[1]], ss[1], add=True)
        for k in (2, 0, 1):
          pltpu.make_async_copy(rows[k], acc.at[prc[k]], ss[k]).wait()
        return _

      lax.fori_loop(0, nblk, _blk, None)

    plsc.subcore_barrier()
    pltpu.sync_copy(acc.at[pl.ds(s * RPTQ, RPTQ)],
                    agg_hbm.at[q, pl.ds(s * RPTQ, RPTQ)])
    plsc.subcore_barrier()


_sc_scatter = functools.partial(
    pl.kernel,
    out_type=jax.ShapeDtypeStruct((4, NPADQ, 2 * H), jnp.float32),
    mesh=_SC_MESH,
    compiler_params=pltpu.CompilerParams(needs_layout_passes=False),
    scratch_types=(
        [pltpu.VMEM_SHARED((NPADQ, 2 * H), jnp.float32)]
        + [pltpu.VMEM((BLKE,), jnp.int32)] * 2
        + [pltpu.VMEM((CHK,), jnp.int32)] * 6
        + [pltpu.VMEM((CHK, 2 * H), jnp.float32)] * 3
        + [pltpu.VMEM((16,), jnp.int32)]
        + [pltpu.SemaphoreType.DMA] * 6
    ),
)(_scat_body)


def _lohi_store(p, h, out_ref):
  z = jnp.zeros_like(h)
  out_ref[:, 0:H] = jnp.where(p == 0, h, z)
  out_ref[:, H:2 * H] = jnp.where(p == 0, z, h)


def _emb_body(x_ref, w_ref, b_ref, h_ref):
  h = (jax.lax.dot_general(x_ref[...], w_ref[...], (((1,), (0,)), ((), ())),
                           preferred_element_type=jnp.float32)
       + b_ref[...])
  _lohi_store(pl.program_id(0), h, h_ref)


def _emb(xp, wp, b):
  return pl.pallas_call(
      _emb_body,
      grid=(2, NSTEPS),
      in_specs=[
          pl.BlockSpec((ROWBLK, 16), lambda p, i: (i, 0)),
          pl.BlockSpec((16, H), lambda p, i: (0, 0)),
          pl.BlockSpec((1, H), lambda p, i: (0, 0)),
      ],
      out_specs=pl.BlockSpec((ROWBLK, 2 * H), lambda p, i: (p * NSTEPS + i, 0)),
      out_shape=jax.ShapeDtypeStruct((2 * N, 2 * H), jnp.float32),
  )(xp, wp, b)


def _stats_body(scal_ref, h_ref, agg_ref, a_ref, s1_ref, g_ref, s1_acc, g_acc):
  i = pl.program_id(0)
  a = scal_ref[0, 0] * h_ref[:, 0:H] + agg_ref[...]
  a_ref[...] = a

  @pl.when(i == 0)
  def _():
    s1_acc[...] = jnp.zeros_like(s1_acc)
    g_acc[...] = jnp.zeros_like(g_acc)

  s1_acc[...] += jnp.sum(a, axis=0, keepdims=True)
  g_acc[...] += jax.lax.dot_general(a, a, (((0,), (0,)), ((), ())),
                                    preferred_element_type=jnp.float32)

  @pl.when(i == NSTEPS - 1)
  def _():
    s1_ref[...] = s1_acc[...]
    g_ref[...] = g_acc[...]


def _stats(scal, h, agg):
  return pl.pallas_call(
      _stats_body,
      grid=(NSTEPS,),
      in_specs=[
          pl.BlockSpec((1, 1), lambda i: (0, 0)),
          pl.BlockSpec((ROWBLK, 2 * H), lambda i: (i, 0)),
          pl.BlockSpec((ROWBLK, H), lambda i: (i, 0)),
      ],
      out_specs=[
          pl.BlockSpec((ROWBLK, H), lambda i: (i, 0)),
          pl.BlockSpec((1, H), lambda i: (0, 0)),
          pl.BlockSpec((H, H), lambda i: (0, 0)),
      ],
      out_shape=[
          jax.ShapeDtypeStruct((N, H), jnp.float32),
          jax.ShapeDtypeStruct((1, H), jnp.float32),
          jax.ShapeDtypeStruct((H, H), jnp.float32),
      ],
      scratch_shapes=[
          pltpu.VMEM((1, H), jnp.float32),
          pltpu.VMEM((H, H), jnp.float32),
      ],
  )(scal, h, agg)


def _mlp_body(a_ref, s1_ref, g_ref, w1_ref, b1_ref, g1_ref, be1_ref,
              w2_ref, b2_ref, z2_ref, s2_ref, q2_ref, s2_acc, q2_acc):
  i = pl.program_id(0)
  inv_n = 1.0 / N
  w1 = w1_ref[...]
  mu = jax.lax.dot_general(s1_ref[...] * inv_n, w1, (((1,), (0,)), ((), ())),
                           preferred_element_type=jnp.float32)
  gw = jax.lax.dot_general(g_ref[...] * inv_n, w1, (((1,), (0,)), ((), ())),
                           preferred_element_type=jnp.float32)
  var = jnp.sum(w1 * gw, axis=0, keepdims=True) - mu * mu
  sc1 = g1_ref[...] * jax.lax.rsqrt(var + 1e-5)
  sh1 = be1_ref[...] - mu * sc1

  z1 = jax.lax.dot_general(a_ref[...], w1, (((1,), (0,)), ((), ())),
                           preferred_element_type=jnp.float32)
  u = jnp.maximum(z1 * sc1 + sh1, 0.0)
  z2 = (jax.lax.dot_general(u, w2_ref[...], (((1,), (0,)), ((), ())),
                            preferred_element_type=jnp.float32)
        + b2_ref[...])
  z2_ref[...] = z2

  @pl.when(i == 0)
  def _():
    s2_acc[...] = jnp.zeros_like(s2_acc)
    q2_acc[...] = jnp.zeros_like(q2_acc)

  s2_acc[...] += jnp.sum(z2, axis=0, keepdims=True)
  q2_acc[...] += jnp.sum(z2 * z2, axis=0, keepdims=True)

  @pl.when(i == NSTEPS - 1)
  def _():
    s2_ref[...] = s2_acc[...]
    q2_ref[...] = q2_acc[...]


def _mlp(a, s1, g, w1, b1, g1, be1, w2, b2):
  return pl.pallas_call(
      _mlp_body,
      grid=(NSTEPS,),
      in_specs=[
          pl.BlockSpec((ROWBLK, H), lambda i: (i, 0)),
          pl.BlockSpec((1, H), lambda i: (0, 0)),
          pl.BlockSpec((H, H), lambda i: (0, 0)),
          pl.BlockSpec((H, 2 * H), lambda i: (0, 0)),
          pl.BlockSpec((1, 2 * H), lambda i: (0, 0)),
          pl.BlockSpec((1, 2 * H), lambda i: (0, 0)),
          pl.BlockSpec((1, 2 * H), lambda i: (0, 0)),
          pl.BlockSpec((2 * H, H), lambda i: (0, 0)),
          pl.BlockSpec((1, H), lambda i: (0, 0)),
      ],
      out_specs=[
          pl.BlockSpec((ROWBLK, H), lambda i: (i, 0)),
          pl.BlockSpec((1, H), lambda i: (0, 0)),
          pl.BlockSpec((1, H), lambda i: (0, 0)),
      ],
      out_shape=[
          jax.ShapeDtypeStruct((N, H), jnp.float32),
          jax.ShapeDtypeStruct((1, H), jnp.float32),
          jax.ShapeDtypeStruct((1, H), jnp.float32),
      ],
      scratch_shapes=[
          pltpu.VMEM((1, H), jnp.float32),
          pltpu.VMEM((1, H), jnp.float32),
      ],
  )(a, s1, g, w1, b1, g1, be1, w2, b2)


def _norm_body(z2_ref, s2_ref, q2_ref, g2_ref, be2_ref, h_ref):
  inv_n = 1.0 / N
  mean = s2_ref[...] * inv_n
  var = q2_ref[...] * inv_n - mean * mean
  sc = g2_ref[...] * jax.lax.rsqrt(var + 1e-5)
  sh = be2_ref[...] - mean * sc
  h = jnp.maximum(z2_ref[...] * sc + sh, 0.0)
  _lohi_store(pl.program_id(0), h, h_ref)


def _norm(z2, s2, q2, g2, be2):
  return pl.pallas_call(
      _norm_body,
      grid=(2, NSTEPS),
      in_specs=[
          pl.BlockSpec((ROWBLK, H), lambda p, i: (i, 0)),
          pl.BlockSpec((1, H), lambda p, i: (0, 0)),
          pl.BlockSpec((1, H), lambda p, i: (0, 0)),
          pl.BlockSpec((1, H), lambda p, i: (0, 0)),
          pl.BlockSpec((1, H), lambda p, i: (0, 0)),
      ],
      out_specs=pl.BlockSpec((ROWBLK, 2 * H), lambda p, i: (p * NSTEPS + i, 0)),
      out_shape=jax.ShapeDtypeStruct((2 * N, 2 * H), jnp.float32),
  )(z2, s2, q2, g2, be2)


def _pool_body(b_ref, h_ref, out_ref, sum_acc, cnt_acc):
  i = pl.program_id(0)

  @pl.when(i == 0)
  def _():
    sum_acc[...] = jnp.zeros_like(sum_acc)
    cnt_acc[...] = jnp.zeros_like(cnt_acc)

  gid = b_ref[0, 0, :]
  onehot = (gid[:, None] ==
            lax.broadcasted_iota(jnp.int32, (ROWBLK, B), 1)
            ).astype(jnp.float32)
  sum_acc[...] += jax.lax.dot_general(
      onehot, h_ref[:, 0:H], (((0,), (0,)), ((), ())),
      preferred_element_type=jnp.float32)
  cnt_acc[...] += jnp.sum(onehot, axis=0, keepdims=True)

  @pl.when(i == NSTEPS - 1)
  def _():
    cnt = jnp.maximum(cnt_acc[...], 1.0)
    inv = (1.0 / cnt)[0, :]
    out_ref[...] = sum_acc[...] * inv[:, None]


def _pool(batch3, h):
  return pl.pallas_call(
      _pool_body,
      grid=(NSTEPS,),
      in_specs=[
          pl.BlockSpec((1, 1, ROWBLK), lambda i: (i, 0, 0)),
          pl.BlockSpec((ROWBLK, 2 * H), lambda i: (i, 0)),
      ],
      out_specs=pl.BlockSpec((B, H), lambda i: (0, 0)),
      out_shape=jax.ShapeDtypeStruct((B, H), jnp.float32),
      scratch_shapes=[
          pltpu.VMEM((B, H), jnp.float32),
          pltpu.VMEM((1, B), jnp.float32),
      ],
  )(batch3, h)


def kernel(x, edge_index, batch, W_emb, b_emb, eps, W1, b1, g1, be1,
           W2, b2, g2, be2):
  xp = jnp.pad(x, ((0, 0), (0, 16 - x.shape[1])))
  wp = jnp.pad(W_emb, ((0, 16 - W_emb.shape[0]), (0, 0)))
  src = jnp.pad(edge_index[0], (0, 16))
  dst = jnp.pad(edge_index[1], (0, 16))
  batch3 = batch.reshape(NSTEPS, 1, ROWBLK)

  gi, pr, nb = _sc_partition(src, dst)
  h = _emb(xp, wp, b_emb.reshape(1, H))
  for i in range(4):
    agg = _sc_scatter(h, gi, pr, nb)[:, :QP, :].reshape(N, H)
    scal = (1.0 + eps[i]).reshape(1, 1)
    a, s1, gmat = _stats(scal, h, agg)
    z2, s2, q2 = _mlp(a, s1, gmat, W1[i], b1[i].reshape(1, 2 * H),
                      g1[i].reshape(1, 2 * H), be1[i].reshape(1, 2 * H),
                      W2[i], b2[i].reshape(1, H))
    h = _norm(z2, s2, q2, g2[i].reshape(1, H), be2[i].reshape(1, H))
  return _pool(batch3, h)
